# baseline (speedup 1.0000x reference)
import functools
import math

import jax
import jax.numpy as jnp
import numpy as np
from jax.experimental import pallas as pl
from jax.experimental.pallas import tpu as pltpu

_L = 2048
_D = 128
_H = 16
_NTOP = 38
_NPAD = 40
_QCHUNK = 512


def _threefry2x32(k1, k2, x0, x1):
    def rotl(x, r):
        return (x << np.uint32(r)) | (x >> np.uint32(32 - r))
    ks = [np.uint32(k1), np.uint32(k2),
          np.uint32(k1) ^ np.uint32(k2) ^ np.uint32(0x1BD11BDA)]
    x = [x0.astype(np.uint32) + ks[0], x1.astype(np.uint32) + ks[1]]
    rounds = [[13, 15, 26, 6], [17, 29, 16, 24]]
    for i in range(5):
        for r in rounds[i % 2]:
            x[0] = x[0] + x[1]
            x[1] = rotl(x[1], r) ^ x[0]
        x[0] = x[0] + ks[(i + 1) % 3]
        x[1] = x[1] + ks[(i + 2) % 3] + np.uint32(i + 1)
    return x[0], x[1]


def _build_sample_weights() -> np.ndarray:
    old = np.seterr(over="ignore")
    b1, b2 = _threefry2x32(0, 42, np.zeros(2, np.uint32),
                           np.arange(2, dtype=np.uint32))
    n = _L * _NTOP
    hb1, hb2 = _threefry2x32(b1[1], b2[1], np.zeros(n, np.uint32),
                             np.arange(n, dtype=np.uint32))
    np.seterr(**old)
    idxs = ((hb1 ^ hb2) % np.uint32(_L)).astype(np.int32).reshape(_L, _NTOP)
    counts = np.zeros((_L, _L), np.float32)
    np.add.at(counts, (np.arange(_L)[:, None], idxs), 1.0)
    return np.ascontiguousarray(counts.T) / np.float32(_NTOP)


_W_NP = _build_sample_weights()


def _score_topk_body(q_ref, k_ref, wt_ref, idx_ref):
    flat = jax.lax.broadcasted_iota(jnp.int32, (1, _L), 1)
    slot = jax.lax.broadcasted_iota(jnp.int32, (1, _NPAD), 1)
    for h in range(_H):
        kh = k_ref[h]
        m_parts = []
        for c in range(_L // _QCHUNK):
            qc = q_ref[h, c * _QCHUNK:(c + 1) * _QCHUNK, :]
            st = jax.lax.dot_general(
                kh, qc, (((1,), (1,)), ((), ())),
                preferred_element_type=jnp.float32)
            wc = wt_ref[:, c * _QCHUNK:(c + 1) * _QCHUNK]
            mx = jnp.max(jnp.where(wc > 0.0, st, -1e30), axis=0, keepdims=True)
            mn = jnp.sum(st * wc, axis=0, keepdims=True)
            m_parts.append(mx - mn)
        mrow = jnp.concatenate(m_parts, axis=1)

        def body(i, carry):
            m, acc = carry
            cur = jnp.max(m)
            sel = jnp.min(jnp.where(m == cur, flat, jnp.int32(2 * _L)))
            m = jnp.where(flat == sel, -1e30, m)
            acc = jnp.where(slot == i, sel, acc)
            return m, acc

        _, acc = jax.lax.fori_loop(
            0, _NTOP, body, (mrow, jnp.zeros((1, _NPAD), jnp.int32)))
        idx_ref[h] = acc


def _attn_scatter_body(idx_sref, q_ref, k_ref, v_ref, out_ref):
    h = pl.program_id(0)
    kh = k_ref[0]
    vh = v_ref[0]
    rows = [q_ref[0, pl.ds(idx_sref[h, i], 1), :] for i in range(_NTOP)]
    qred = jnp.concatenate(rows, axis=0)
    s = jax.lax.dot_general(
        qred, kh, (((1,), (1,)), ((), ())),
        preferred_element_type=jnp.float32) * (1.0 / math.sqrt(_D))
    mx = jnp.max(s, axis=1, keepdims=True)
    e = jnp.exp(s - mx)
    p = e / jnp.sum(e, axis=1, keepdims=True)
    ctx = jax.lax.dot_general(
        p, vh, (((1,), (0,)), ((), ())),
        preferred_element_type=jnp.float32)
    vmean = jnp.mean(vh, axis=0, keepdims=True)
    out_ref[0] = jnp.broadcast_to(vmean, (_L, _D))
    for i in range(_NTOP):
        out_ref[0, pl.ds(idx_sref[h, i], 1), :] = ctx[i:i + 1, :]


@jax.jit
def kernel(q, k, v):
    b, h, l, d = q.shape
    q3, k3, v3 = q[0], k[0], v[0]
    w = jnp.asarray(_W_NP)

    idx = pl.pallas_call(
        _score_topk_body,
        out_shape=jax.ShapeDtypeStruct((_H, 1, _NPAD), jnp.int32),
    )(q3, k3, w)

    grid_spec = pltpu.PrefetchScalarGridSpec(
        num_scalar_prefetch=1,
        grid=(_H,),
        in_specs=[
            pl.BlockSpec((1, _L, _D), lambda hh, idx_s: (hh, 0, 0)),
            pl.BlockSpec((1, _L, _D), lambda hh, idx_s: (hh, 0, 0)),
            pl.BlockSpec((1, _L, _D), lambda hh, idx_s: (hh, 0, 0)),
        ],
        out_specs=pl.BlockSpec((1, _L, _D), lambda hh, idx_s: (hh, 0, 0)),
    )
    out = pl.pallas_call(
        _attn_scatter_body,
        grid_spec=grid_spec,
        out_shape=jax.ShapeDtypeStruct((_H, _L, _D), jnp.float32),
    )(idx.reshape(_H, _NPAD), q3, k3, v3)
    return out[None]


# speedup vs baseline: 2.2908x; 2.2908x over previous
import functools
import math

import jax
import jax.numpy as jnp
import numpy as np
from jax.experimental import pallas as pl
from jax.experimental.pallas import tpu as pltpu
from jax.experimental.pallas import tpu_sc as plsc

_L = 2048
_D = 128
_H = 16
_NTOP = 38
_NPAD = 40
_QCHUNK = 512


def _threefry2x32(k1, k2, x0, x1):
    def rotl(x, r):
        return (x << np.uint32(r)) | (x >> np.uint32(32 - r))
    ks = [np.uint32(k1), np.uint32(k2),
          np.uint32(k1) ^ np.uint32(k2) ^ np.uint32(0x1BD11BDA)]
    x = [x0.astype(np.uint32) + ks[0], x1.astype(np.uint32) + ks[1]]
    rounds = [[13, 15, 26, 6], [17, 29, 16, 24]]
    for i in range(5):
        for r in rounds[i % 2]:
            x[0] = x[0] + x[1]
            x[1] = rotl(x[1], r) ^ x[0]
        x[0] = x[0] + ks[(i + 1) % 3]
        x[1] = x[1] + ks[(i + 2) % 3] + np.uint32(i + 1)
    return x[0], x[1]


def _build_sample_weights() -> np.ndarray:
    old = np.seterr(over="ignore")
    b1, b2 = _threefry2x32(0, 42, np.zeros(2, np.uint32),
                           np.arange(2, dtype=np.uint32))
    n = _L * _NTOP
    hb1, hb2 = _threefry2x32(b1[1], b2[1], np.zeros(n, np.uint32),
                             np.arange(n, dtype=np.uint32))
    np.seterr(**old)
    idxs = ((hb1 ^ hb2) % np.uint32(_L)).astype(np.int32).reshape(_L, _NTOP)
    counts = np.zeros((_L, _L), np.float32)
    np.add.at(counts, (np.arange(_L)[:, None], idxs), 1.0)
    wt = np.ascontiguousarray(counts.T) / np.float32(_NTOP)
    amask = np.where(wt > 0.0, np.float32(0.0), np.float32(-1e30))
    return wt, amask


_W_NP, _A_NP = _build_sample_weights()


def _score_body(q_ref, k_ref, wt_ref, am_ref, m_ref):
    kh = k_ref[0]
    qc = q_ref[0]
    st = jax.lax.dot_general(
        kh, qc, (((1,), (1,)), ((), ())),
        preferred_element_type=jnp.float32)
    mx = jnp.max(st + am_ref[...], axis=0)
    mn = jnp.sum(st * wt_ref[...], axis=0)
    m_ref[...] = mx - mn


def _vmax_tree(vs):
    vs = list(vs)
    while len(vs) > 1:
        nxt = [jnp.maximum(vs[i], vs[i + 1]) for i in range(0, len(vs) - 1, 2)]
        if len(vs) % 2:
            nxt.append(vs[-1])
        vs = nxt
    return vs[0]


def _extract_reduce(vec, op):
    vals = [vec[i] for i in range(16)]
    while len(vals) > 1:
        vals = [op(vals[i], vals[i + 1]) for i in range(0, len(vals), 2)]
    return vals[0]


def _sc_topk_body(m_hbm, idx_hbm, m_v, idx_v):
    lane = jax.lax.broadcasted_iota(jnp.int32, (16,), 0)
    wid = jax.lax.axis_index("s") * 2 + jax.lax.axis_index("c")

    @pl.when(wid < _H)
    def _():
        pltpu.sync_copy(m_hbm.at[pl.ds(wid * _L, _L)], m_v)
        for g in range(3):
            idx_v[pl.ds(g * 16, 16)] = jnp.zeros((16,), jnp.int32)
        G = []
        for g in range(8):
            cols = [m_v[pl.ds((g * 16 + c) * 16, 16)] for c in range(16)]
            G.append(_vmax_tree(cols))

        def body(it, carry):
            G = list(carry)
            r = _vmax_tree(G)
            gmax = _extract_reduce(r, jnp.maximum)
            gs = jnp.full((16,), gmax, jnp.float32)
            gf = jnp.full((16,), jnp.int32(99), jnp.int32)
            for g in range(8):
                gf = jnp.minimum(gf, jnp.where(G[g] == gs, g, 99))
            gstar = _extract_reduce(gf, jnp.minimum)
            cand = jnp.full((16,), jnp.int32(9999), jnp.int32)
            vcs = []
            for c in range(16):
                off = pl.multiple_of(gstar * 256 + c * 16, 16)
                vc = m_v[pl.ds(off, 16)]
                vcs.append(vc)
                cand = jnp.minimum(cand,
                                   jnp.where(vc == gs, c * 16 + lane, 9999))
            rel = _extract_reduce(cand, jnp.minimum)
            gidx = gstar * 256 + rel
            ioff = pl.multiple_of((it // 16) * 16, 16)
            iword = idx_v[pl.ds(ioff, 16)]
            idx_v[pl.ds(ioff, 16)] = jnp.where(lane == it % 16, gidx, iword)
            cstar = rel // 16
            ln = rel % 16
            moff = pl.multiple_of(gstar * 256 + cstar * 16, 16)
            v2 = jnp.where(lane == ln, -1e30, m_v[pl.ds(moff, 16)])
            m_v[pl.ds(moff, 16)] = v2
            eff = [jnp.where(jnp.int32(c) == cstar, v2, vcs[c])
                   for c in range(16)]
            gnew = _vmax_tree(eff)
            G = [jnp.where(jnp.int32(g) == gstar, gnew, G[g])
                 for g in range(8)]
            return tuple(G)

        jax.lax.fori_loop(0, _NTOP, body, tuple(G))
        pltpu.sync_copy(idx_v.at[pl.ds(0, _NPAD)],
                        idx_hbm.at[pl.ds(wid * _NPAD, _NPAD)])


def _attn_scatter_body(idx_sref, q_ref, k_ref, v_ref, out_ref):
    h = pl.program_id(0)
    kh = k_ref[0]
    vh = v_ref[0]
    rows = [q_ref[0, pl.ds(idx_sref[h, i], 1), :] for i in range(_NTOP)]
    qred = jnp.concatenate(rows, axis=0)
    s = jax.lax.dot_general(
        qred, kh, (((1,), (1,)), ((), ())),
        preferred_element_type=jnp.float32) * (1.0 / math.sqrt(_D))
    mx = jnp.max(s, axis=1, keepdims=True)
    e = jnp.exp(s - mx)
    p = e / jnp.sum(e, axis=1, keepdims=True)
    ctx = jax.lax.dot_general(
        p, vh, (((1,), (0,)), ((), ())),
        preferred_element_type=jnp.float32)
    vmean = jnp.mean(vh, axis=0, keepdims=True)
    out_ref[0] = jnp.broadcast_to(vmean, (_L, _D))
    for i in range(_NTOP):
        out_ref[0, pl.ds(idx_sref[h, i], 1), :] = ctx[i:i + 1, :]


@jax.jit
def kernel(q, k, v):
    b, h, l, d = q.shape
    q3, k3, v3 = q[0], k[0], v[0]
    w = jnp.asarray(_W_NP)
    amask = jnp.asarray(_A_NP)

    m = pl.pallas_call(
        _score_body,
        grid=(_L // _QCHUNK, _H),
        in_specs=[
            pl.BlockSpec((1, _QCHUNK, _D), lambda c, hh: (hh, c, 0)),
            pl.BlockSpec((1, _L, _D), lambda c, hh: (hh, 0, 0)),
            pl.BlockSpec((_L, _QCHUNK), lambda c, hh: (0, c)),
            pl.BlockSpec((_L, _QCHUNK), lambda c, hh: (0, c)),
        ],
        out_specs=pl.BlockSpec((_QCHUNK,), lambda c, hh: (hh * (_L // _QCHUNK) + c)),
        out_shape=jax.ShapeDtypeStruct((_H * _L,), jnp.float32),
    )(q3, k3, w, amask)

    sc_topk = pl.kernel(
        _sc_topk_body,
        out_type=jax.ShapeDtypeStruct((_H * _NPAD,), jnp.int32),
        mesh=plsc.VectorSubcoreMesh(core_axis_name="c", subcore_axis_name="s"),
        scratch_types=[
            pltpu.VMEM((_L,), jnp.float32),
            pltpu.VMEM((48,), jnp.int32),
        ],
    )
    idx = sc_topk(m).reshape(_H, _NPAD)

    grid_spec = pltpu.PrefetchScalarGridSpec(
        num_scalar_prefetch=1,
        grid=(_H,),
        in_specs=[
            pl.BlockSpec((1, _L, _D), lambda hh, idx_s: (hh, 0, 0)),
            pl.BlockSpec((1, _L, _D), lambda hh, idx_s: (hh, 0, 0)),
            pl.BlockSpec((1, _L, _D), lambda hh, idx_s: (hh, 0, 0)),
        ],
        out_specs=pl.BlockSpec((1, _L, _D), lambda hh, idx_s: (hh, 0, 0)),
    )
    out = pl.pallas_call(
        _attn_scatter_body,
        grid_spec=grid_spec,
        out_shape=jax.ShapeDtypeStruct((_H, _L, _D), jnp.float32),
    )(idx, q3, k3, v3)
    return out[None]


# speedup vs baseline: 2.9238x; 1.2763x over previous
import functools
import math

import jax
import jax.numpy as jnp
import numpy as np
from jax.experimental import pallas as pl
from jax.experimental.pallas import tpu as pltpu
from jax.experimental.pallas import tpu_sc as plsc

_L = 2048
_D = 128
_H = 16
_NTOP = 38
_NPAD = 40
_QCHUNK = 512


def _threefry2x32(k1, k2, x0, x1):
    def rotl(x, r):
        return (x << np.uint32(r)) | (x >> np.uint32(32 - r))
    ks = [np.uint32(k1), np.uint32(k2),
          np.uint32(k1) ^ np.uint32(k2) ^ np.uint32(0x1BD11BDA)]
    x = [x0.astype(np.uint32) + ks[0], x1.astype(np.uint32) + ks[1]]
    rounds = [[13, 15, 26, 6], [17, 29, 16, 24]]
    for i in range(5):
        for r in rounds[i % 2]:
            x[0] = x[0] + x[1]
            x[1] = rotl(x[1], r) ^ x[0]
        x[0] = x[0] + ks[(i + 1) % 3]
        x[1] = x[1] + ks[(i + 2) % 3] + np.uint32(i + 1)
    return x[0], x[1]


def _build_sample_weights() -> np.ndarray:
    old = np.seterr(over="ignore")
    b1, b2 = _threefry2x32(0, 42, np.zeros(2, np.uint32),
                           np.arange(2, dtype=np.uint32))
    n = _L * _NTOP
    hb1, hb2 = _threefry2x32(b1[1], b2[1], np.zeros(n, np.uint32),
                             np.arange(n, dtype=np.uint32))
    np.seterr(**old)
    idxs = ((hb1 ^ hb2) % np.uint32(_L)).astype(np.int32).reshape(_L, _NTOP)
    counts = np.zeros((_L, _L), np.float32)
    np.add.at(counts, (np.arange(_L)[:, None], idxs), 1.0)
    wt = np.ascontiguousarray(counts.T) / np.float32(_NTOP)
    amask = np.where(wt > 0.0, np.float32(0.0), np.float32(-1e30))
    return wt, amask


_W_NP, _A_NP = _build_sample_weights()


def _score_body(q_ref, k_ref, wt_ref, am_ref, m_ref):
    kh = k_ref[0]
    m_parts = []
    for c in range(_L // _QCHUNK):
        qc = q_ref[0, c * _QCHUNK:(c + 1) * _QCHUNK, :]
        st = jax.lax.dot_general(
            kh, qc, (((1,), (1,)), ((), ())),
            preferred_element_type=jnp.float32)
        wc = wt_ref[:, c * _QCHUNK:(c + 1) * _QCHUNK]
        ac = am_ref[:, c * _QCHUNK:(c + 1) * _QCHUNK]
        mx = jnp.max(st + ac, axis=0, keepdims=True)
        mn = jnp.sum(st * wc, axis=0, keepdims=True)
        m_parts.append(mx - mn)
    m_ref[...] = jnp.concatenate(m_parts, axis=1)[0]


def _vmax_tree(vs):
    vs = list(vs)
    while len(vs) > 1:
        nxt = [jnp.maximum(vs[i], vs[i + 1]) for i in range(0, len(vs) - 1, 2)]
        if len(vs) % 2:
            nxt.append(vs[-1])
        vs = nxt
    return vs[0]


def _extract_reduce(vec, op):
    vals = [vec[i] for i in range(16)]
    while len(vals) > 1:
        vals = [op(vals[i], vals[i + 1]) for i in range(0, len(vals), 2)]
    return vals[0]


def _sc_topk_body(m_hbm, idx_hbm, m_v, idx_v):
    lane = jax.lax.broadcasted_iota(jnp.int32, (16,), 0)
    wid = jax.lax.axis_index("s") * 2 + jax.lax.axis_index("c")

    @pl.when(wid < _H)
    def _():
        pltpu.sync_copy(m_hbm.at[pl.ds(wid * _L, _L)], m_v)
        for g in range(3):
            idx_v[pl.ds(g * 16, 16)] = jnp.zeros((16,), jnp.int32)
        G = []
        for g in range(8):
            cols = [m_v[pl.ds((g * 16 + c) * 16, 16)] for c in range(16)]
            G.append(_vmax_tree(cols))

        def body(it, carry):
            G = list(carry)
            r = _vmax_tree(G)
            gmax = _extract_reduce(r, jnp.maximum)
            gs = jnp.full((16,), gmax, jnp.float32)
            gf = jnp.full((16,), jnp.int32(99), jnp.int32)
            for g in range(8):
                gf = jnp.minimum(gf, jnp.where(G[g] == gs, g, 99))
            gstar = _extract_reduce(gf, jnp.minimum)
            cand = jnp.full((16,), jnp.int32(9999), jnp.int32)
            vcs = []
            for c in range(16):
                off = pl.multiple_of(gstar * 256 + c * 16, 16)
                vc = m_v[pl.ds(off, 16)]
                vcs.append(vc)
                cand = jnp.minimum(cand,
                                   jnp.where(vc == gs, c * 16 + lane, 9999))
            rel = _extract_reduce(cand, jnp.minimum)
            gidx = gstar * 256 + rel
            ioff = pl.multiple_of((it // 16) * 16, 16)
            iword = idx_v[pl.ds(ioff, 16)]
            idx_v[pl.ds(ioff, 16)] = jnp.where(lane == it % 16, gidx, iword)
            cstar = rel // 16
            ln = rel % 16
            moff = pl.multiple_of(gstar * 256 + cstar * 16, 16)
            v2 = jnp.where(lane == ln, -1e30, m_v[pl.ds(moff, 16)])
            m_v[pl.ds(moff, 16)] = v2
            eff = [jnp.where(jnp.int32(c) == cstar, v2, vcs[c])
                   for c in range(16)]
            gnew = _vmax_tree(eff)
            G = [jnp.where(jnp.int32(g) == gstar, gnew, G[g])
                 for g in range(8)]
            return tuple(G)

        jax.lax.fori_loop(0, _NTOP, body, tuple(G))
        pltpu.sync_copy(idx_v.at[pl.ds(0, _NPAD)],
                        idx_hbm.at[pl.ds(wid * _NPAD, _NPAD)])


def _attn_scatter_body(idx_sref, q_ref, k_ref, v_ref, out_ref):
    h = pl.program_id(0)
    kh = k_ref[0]
    vh = v_ref[0]
    rows = [q_ref[0, pl.ds(idx_sref[h, i], 1), :] for i in range(_NTOP)]
    qred = jnp.concatenate(rows, axis=0)
    s = jax.lax.dot_general(
        qred, kh, (((1,), (1,)), ((), ())),
        preferred_element_type=jnp.float32) * (1.0 / math.sqrt(_D))
    mx = jnp.max(s, axis=1, keepdims=True)
    e = jnp.exp(s - mx)
    p = e / jnp.sum(e, axis=1, keepdims=True)
    ctx = jax.lax.dot_general(
        p, vh, (((1,), (0,)), ((), ())),
        preferred_element_type=jnp.float32)
    vmean = jnp.mean(vh, axis=0, keepdims=True)
    out_ref[0] = jnp.broadcast_to(vmean, (_L, _D))
    for i in range(_NTOP):
        out_ref[0, pl.ds(idx_sref[h, i], 1), :] = ctx[i:i + 1, :]


@jax.jit
def kernel(q, k, v):
    b, h, l, d = q.shape
    q3, k3, v3 = q[0], k[0], v[0]
    w = jnp.asarray(_W_NP)
    amask = jnp.asarray(_A_NP)

    m = pl.pallas_call(
        _score_body,
        grid=(_H,),
        in_specs=[
            pl.BlockSpec((1, _L, _D), lambda hh: (hh, 0, 0)),
            pl.BlockSpec((1, _L, _D), lambda hh: (hh, 0, 0)),
            pl.BlockSpec((_L, _L), lambda hh: (0, 0)),
            pl.BlockSpec((_L, _L), lambda hh: (0, 0)),
        ],
        out_specs=pl.BlockSpec((_L,), lambda hh: (hh,)),
        out_shape=jax.ShapeDtypeStruct((_H * _L,), jnp.float32),
    )(q3, k3, w, amask)

    sc_topk = pl.kernel(
        _sc_topk_body,
        out_type=jax.ShapeDtypeStruct((_H * _NPAD,), jnp.int32),
        mesh=plsc.VectorSubcoreMesh(core_axis_name="c", subcore_axis_name="s"),
        scratch_types=[
            pltpu.VMEM((_L,), jnp.float32),
            pltpu.VMEM((48,), jnp.int32),
        ],
    )
    idx = sc_topk(m).reshape(_H, _NPAD)

    grid_spec = pltpu.PrefetchScalarGridSpec(
        num_scalar_prefetch=1,
        grid=(_H,),
        in_specs=[
            pl.BlockSpec((1, _L, _D), lambda hh, idx_s: (hh, 0, 0)),
            pl.BlockSpec((1, _L, _D), lambda hh, idx_s: (hh, 0, 0)),
            pl.BlockSpec((1, _L, _D), lambda hh, idx_s: (hh, 0, 0)),
        ],
        out_specs=pl.BlockSpec((1, _L, _D), lambda hh, idx_s: (hh, 0, 0)),
    )
    out = pl.pallas_call(
        _attn_scatter_body,
        grid_spec=grid_spec,
        out_shape=jax.ShapeDtypeStruct((_H, _L, _D), jnp.float32),
    )(idx, q3, k3, v3)
    return out[None]
